# Initial kernel scaffold; baseline (speedup 1.0000x reference)
import functools

import jax
import jax.numpy as jnp
from jax import lax
from jax.experimental import pallas as pl
from jax.experimental.pallas import tpu as pltpu
from jax.experimental.pallas import tpu_sc as plsc

N = 10000
E = 320000
D = 128
NC = 2
NS = 16
NW = NC * NS
EPW = E // NW
CHUNK = 80
NCHUNK = EPW // CHUNK
RPT = N // NS
ZROWS = 125


def _sc_aggregate(x, src_r, dst_r):
    mesh = plsc.VectorSubcoreMesh(core_axis_name="c", subcore_axis_name="s")

    @functools.partial(
        pl.kernel,
        out_type=[
            jax.ShapeDtypeStruct((NC, N, D), jnp.float32),
            jax.ShapeDtypeStruct((NC, N, 16), jnp.float32),
        ],
        mesh=mesh,
        scratch_types=[
            pltpu.VMEM((NCHUNK, CHUNK), jnp.int32),
            pltpu.VMEM((NCHUNK, CHUNK), jnp.int32),
            pltpu.VMEM((CHUNK, D), jnp.float32),
            pltpu.VMEM((CHUNK, 16), jnp.float32),
            pltpu.VMEM((ZROWS, D), jnp.float32),
            pltpu.VMEM((ZROWS, 16), jnp.float32),
            pltpu.VMEM_SHARED((N, D), jnp.float32),
            pltpu.VMEM_SHARED((N, 16), jnp.float32),
            pltpu.SemaphoreType.DMA,
        ],
    )
    def body(x_hbm, src_hbm, dst_hbm, agg_out, deg_out,
             srcb, dstb, rows, ones, zer, zer16, agg_sh, deg_sh, sem):
        c = lax.axis_index("c")
        s = lax.axis_index("s")
        wid = s * NC + c

        zv = jnp.zeros((16,), jnp.float32)
        ov = jnp.ones((16,), jnp.float32)

        def zfill(i, carry):
            zer[i // 8, pl.ds((i % 8) * 16, 16)] = zv
            return carry
        lax.fori_loop(0, ZROWS * 8, zfill, 0)

        def z16fill(i, carry):
            zer16[i] = zv
            return carry
        lax.fori_loop(0, ZROWS, z16fill, 0)

        def ofill(i, carry):
            ones[i] = ov
            return carry
        lax.fori_loop(0, CHUNK, ofill, 0)

        for j in range(RPT // ZROWS):
            base = s * RPT + j * ZROWS
            pltpu.sync_copy(zer, agg_sh.at[pl.ds(base, ZROWS)])
            pltpu.sync_copy(zer16, deg_sh.at[pl.ds(base, ZROWS)])

        pltpu.sync_copy(src_hbm.at[wid], srcb)
        pltpu.sync_copy(dst_hbm.at[wid], dstb)

        plsc.subcore_barrier()

        def step(t, carry):
            pltpu.async_copy(x_hbm.at[srcb.at[t]], rows, sem).wait()
            pltpu.sync_copy(rows, agg_sh.at[dstb.at[t]], add=True)
            pltpu.sync_copy(ones, deg_sh.at[dstb.at[t]], add=True)
            return carry
        lax.fori_loop(0, NCHUNK, step, 0)

        plsc.subcore_barrier()

        pltpu.sync_copy(agg_sh.at[pl.ds(s * RPT, RPT)],
                        agg_out.at[c, pl.ds(s * RPT, RPT)])
        pltpu.sync_copy(deg_sh.at[pl.ds(s * RPT, RPT)],
                        deg_out.at[c, pl.ds(s * RPT, RPT)])

    return body(x, src_r, dst_r)


BN = 1000


def _tc_combine(aggsum, degbuf, x, wl_t, wr_t, b_row):
    def body(agg_ref, deg_ref, x_ref, wl_ref, wr_ref, b_ref, o_ref):
        a = agg_ref[0] + agg_ref[1]
        d = deg_ref[0, :, 0:1] + deg_ref[1, :, 0:1]
        agg = a * (1.0 / jnp.maximum(d, 1.0))
        out = jnp.dot(agg, wl_ref[...], preferred_element_type=jnp.float32)
        out = out + jnp.dot(x_ref[...], wr_ref[...],
                            preferred_element_type=jnp.float32)
        out = out + b_ref[...]
        o_ref[...] = jnp.maximum(out, 0.0)

    return pl.pallas_call(
        body,
        grid=(N // BN,),
        in_specs=[
            pl.BlockSpec((NC, BN, D), lambda i: (0, i, 0)),
            pl.BlockSpec((NC, BN, 16), lambda i: (0, i, 0)),
            pl.BlockSpec((BN, D), lambda i: (i, 0)),
            pl.BlockSpec((D, D), lambda i: (0, 0)),
            pl.BlockSpec((D, D), lambda i: (0, 0)),
            pl.BlockSpec((1, D), lambda i: (0, 0)),
        ],
        out_specs=pl.BlockSpec((BN, D), lambda i: (i, 0)),
        out_shape=jax.ShapeDtypeStruct((N, D), jnp.float32),
    )(aggsum, degbuf, x, wl_t, wr_t, b_row)


@jax.jit
def kernel(x, edge_index, W_l, b_l, W_r):
    src_r = edge_index[0].reshape(NW, NCHUNK, CHUNK)
    dst_r = edge_index[1].reshape(NW, NCHUNK, CHUNK)
    aggsum, degbuf = _sc_aggregate(x, src_r, dst_r)
    return _tc_combine(aggsum, degbuf, x, W_l.T, W_r.T, b_l.reshape(1, D))


# speedup vs baseline: 6.0974x; 6.0974x over previous
import functools

import jax
import jax.numpy as jnp
from jax import lax
from jax.experimental import pallas as pl
from jax.experimental.pallas import tpu as pltpu
from jax.experimental.pallas import tpu_sc as plsc

N = 10000
E = 320000
D = 128
DH = D // 2
NC = 2
NS = 16
EPT = E // NS
CHUNK = 80
NCHUNK = EPT // CHUNK
NP = 10240
RPT = NP // NS


def _sc_aggregate(x0, x1, src_r, dst_r):
    mesh = plsc.VectorSubcoreMesh(core_axis_name="c", subcore_axis_name="s")

    @functools.partial(
        pl.kernel,
        out_type=[
            jax.ShapeDtypeStruct((NC, NP, DH), jnp.float32),
            jax.ShapeDtypeStruct((NP, 16), jnp.float32),
        ],
        mesh=mesh,
        compiler_params=pltpu.CompilerParams(use_tc_tiling_on_sc=False),
        scratch_types=[
            pltpu.VMEM((NCHUNK, CHUNK), jnp.int32),
            pltpu.VMEM((NCHUNK, CHUNK), jnp.int32),
            pltpu.VMEM((CHUNK, DH), jnp.float32),
            pltpu.VMEM((CHUNK, 16), jnp.float32),
            pltpu.VMEM((CHUNK, 16), jnp.float32),
            pltpu.VMEM_SHARED((NP, DH), jnp.float32),
            pltpu.VMEM_SHARED((NP, 16), jnp.float32),
            pltpu.SemaphoreType.DMA,
        ],
    )
    def body(x0_hbm, x1_hbm, src_hbm, dst_hbm, agg_out, deg_out,
             srcb, dstb, rows, ones, zer16, agg_sh, deg_sh, sem):
        c = lax.axis_index("c")
        s = lax.axis_index("s")

        zv = jnp.zeros((16,), jnp.float32)
        ov = jnp.ones((16,), jnp.float32)

        def zfill(i, carry):
            rows[i // 4, pl.ds((i % 4) * 16, 16)] = zv
            return carry
        lax.fori_loop(0, CHUNK * 4, zfill, 0)

        def z16fill(i, carry):
            zer16[i] = zv
            return carry
        lax.fori_loop(0, CHUNK, z16fill, 0)

        def ofill(i, carry):
            ones[i] = ov
            return carry
        lax.fori_loop(0, CHUNK, ofill, 0)

        for j in range(RPT // CHUNK):
            base = s * RPT + j * CHUNK
            pltpu.sync_copy(rows, agg_sh.at[pl.ds(base, CHUNK)])
            pltpu.sync_copy(zer16, deg_sh.at[pl.ds(base, CHUNK)])

        pltpu.sync_copy(src_hbm.at[s], srcb)
        pltpu.sync_copy(dst_hbm.at[s], dstb)

        plsc.subcore_barrier()

        @pl.when(c == 0)
        def _():
            def step(t, carry):
                pltpu.async_copy(x0_hbm.at[srcb.at[t]], rows, sem).wait()
                pltpu.sync_copy(rows, agg_sh.at[dstb.at[t]], add=True)
                pltpu.sync_copy(ones, deg_sh.at[dstb.at[t]], add=True)
                return carry
            lax.fori_loop(0, NCHUNK, step, 0)

        @pl.when(c == 1)
        def _():
            def step(t, carry):
                pltpu.async_copy(x1_hbm.at[srcb.at[t]], rows, sem).wait()
                pltpu.sync_copy(rows, agg_sh.at[dstb.at[t]], add=True)
                return carry
            lax.fori_loop(0, NCHUNK, step, 0)

        plsc.subcore_barrier()

        pltpu.sync_copy(agg_sh.at[pl.ds(s * RPT, RPT)],
                        agg_out.at[c, pl.ds(s * RPT, RPT)])

        @pl.when(c == 0)
        def _():
            pltpu.sync_copy(deg_sh.at[pl.ds(s * RPT, RPT)],
                            deg_out.at[pl.ds(s * RPT, RPT)])

    return body(x0, x1, src_r, dst_r)


BN = 1000


def _tc_combine(aggsum, degbuf, x, wl_t, wr_t, b_row):
    def body(agg_ref, deg_ref, x_ref, wl_ref, wr_ref, b_ref, o_ref):
        inv = 1.0 / jnp.maximum(deg_ref[:, 0:1], 1.0)
        a0 = agg_ref[0] * inv
        a1 = agg_ref[1] * inv
        out = jnp.dot(a0, wl_ref[0:DH, :], preferred_element_type=jnp.float32)
        out = out + jnp.dot(a1, wl_ref[DH:D, :],
                            preferred_element_type=jnp.float32)
        out = out + jnp.dot(x_ref[...], wr_ref[...],
                            preferred_element_type=jnp.float32)
        out = out + b_ref[...]
        o_ref[...] = jnp.maximum(out, 0.0)

    return pl.pallas_call(
        body,
        grid=(N // BN,),
        in_specs=[
            pl.BlockSpec((NC, BN, DH), lambda i: (0, i, 0)),
            pl.BlockSpec((BN, 16), lambda i: (i, 0)),
            pl.BlockSpec((BN, D), lambda i: (i, 0)),
            pl.BlockSpec((D, D), lambda i: (0, 0)),
            pl.BlockSpec((D, D), lambda i: (0, 0)),
            pl.BlockSpec((1, D), lambda i: (0, 0)),
        ],
        out_specs=pl.BlockSpec((BN, D), lambda i: (i, 0)),
        out_shape=jax.ShapeDtypeStruct((N, D), jnp.float32),
    )(aggsum, degbuf, x, wl_t, wr_t, b_row)


@jax.jit
def kernel(x, edge_index, W_l, b_l, W_r):
    x0 = x[:, :DH]
    x1 = x[:, DH:]
    src_r = edge_index[0].reshape(NS, NCHUNK, CHUNK)
    dst_r = edge_index[1].reshape(NS, NCHUNK, CHUNK)
    aggsum, degbuf = _sc_aggregate(x0, x1, src_r, dst_r)
    return _tc_combine(aggsum, degbuf, x, W_l.T, W_r.T, b_l.reshape(1, D))


# speedup vs baseline: 8.5439x; 1.4012x over previous
import functools

import jax
import jax.numpy as jnp
from jax import lax
from jax.experimental import pallas as pl
from jax.experimental.pallas import tpu as pltpu
from jax.experimental.pallas import tpu_sc as plsc

N = 10000
E = 320000
D = 128
DH = D // 2
NC = 2
NS = 16
EPT = E // NS
CHUNK = 80
NCHUNK = EPT // CHUNK
NP = 10240
RPT = NP // NS


def _sc_aggregate(x0, x1, src_r, dst_r):
    mesh = plsc.VectorSubcoreMesh(core_axis_name="c", subcore_axis_name="s")

    @functools.partial(
        pl.kernel,
        out_type=[
            jax.ShapeDtypeStruct((NC, NP, DH), jnp.float32),
            jax.ShapeDtypeStruct((NP, 16), jnp.float32),
        ],
        mesh=mesh,
        compiler_params=pltpu.CompilerParams(use_tc_tiling_on_sc=False),
        scratch_types=[
            pltpu.VMEM((NCHUNK + 2, CHUNK), jnp.int32),
            pltpu.VMEM((NCHUNK, CHUNK), jnp.int32),
            pltpu.VMEM((CHUNK, DH), jnp.float32),
            pltpu.VMEM((CHUNK, DH), jnp.float32),
            pltpu.VMEM((CHUNK, 16), jnp.float32),
            pltpu.VMEM((CHUNK, 16), jnp.float32),
            pltpu.VMEM_SHARED((NP, DH), jnp.float32),
            pltpu.VMEM_SHARED((NP, 16), jnp.float32),
            pltpu.SemaphoreType.DMA,
            pltpu.SemaphoreType.DMA,
        ],
    )
    def body(x0_hbm, x1_hbm, src_hbm, dst_hbm, agg_out, deg_out,
             srcb, dstb, rows0, rows1, ones, zer16, agg_sh, deg_sh,
             sem0, sem1):
        c = lax.axis_index("c")
        s = lax.axis_index("s")

        zv = jnp.zeros((16,), jnp.float32)
        zvi = jnp.zeros((16,), jnp.int32)
        ov = jnp.ones((16,), jnp.float32)

        rows = rows0

        def zfill(i, carry):
            rows0[i // 4, pl.ds((i % 4) * 16, 16)] = zv
            return carry
        lax.fori_loop(0, CHUNK * 4, zfill, 0)

        for r in (NCHUNK, NCHUNK + 1):
            for j in range(CHUNK // 16):
                srcb[r, pl.ds(j * 16, 16)] = zvi

        def z16fill(i, carry):
            zer16[i] = zv
            return carry
        lax.fori_loop(0, CHUNK, z16fill, 0)

        def ofill(i, carry):
            ones[i] = ov
            return carry
        lax.fori_loop(0, CHUNK, ofill, 0)

        for j in range(RPT // CHUNK):
            base = s * RPT + j * CHUNK
            pltpu.sync_copy(rows, agg_sh.at[pl.ds(base, CHUNK)])
            pltpu.sync_copy(zer16, deg_sh.at[pl.ds(base, CHUNK)])

        pltpu.sync_copy(src_hbm.at[s], srcb.at[pl.ds(0, NCHUNK)])
        pltpu.sync_copy(dst_hbm.at[s], dstb)

        plsc.subcore_barrier()

        def run_loop(x_hbm, with_deg):
            pltpu.async_copy(x_hbm.at[srcb.at[0]], rows0, sem0)

            def pair(p, carry):
                t0 = 2 * p
                g1 = pltpu.async_copy(x_hbm.at[srcb.at[t0 + 1]], rows1, sem1)
                pltpu.make_async_copy(x_hbm.at[srcb.at[t0]], rows0,
                                      sem0).wait()
                pltpu.sync_copy(rows0, agg_sh.at[dstb.at[t0]], add=True)
                if with_deg:
                    pltpu.sync_copy(ones, deg_sh.at[dstb.at[t0]], add=True)
                pltpu.async_copy(x_hbm.at[srcb.at[t0 + 2]], rows0, sem0)
                g1.wait()
                pltpu.sync_copy(rows1, agg_sh.at[dstb.at[t0 + 1]], add=True)
                if with_deg:
                    pltpu.sync_copy(ones, deg_sh.at[dstb.at[t0 + 1]],
                                    add=True)
                return carry
            lax.fori_loop(0, NCHUNK // 2, pair, 0)
            pltpu.make_async_copy(x_hbm.at[srcb.at[NCHUNK]], rows0,
                                  sem0).wait()

        @pl.when(c == 0)
        def _():
            run_loop(x0_hbm, True)

        @pl.when(c == 1)
        def _():
            run_loop(x1_hbm, False)

        plsc.subcore_barrier()

        pltpu.sync_copy(agg_sh.at[pl.ds(s * RPT, RPT)],
                        agg_out.at[c, pl.ds(s * RPT, RPT)])

        @pl.when(c == 0)
        def _():
            pltpu.sync_copy(deg_sh.at[pl.ds(s * RPT, RPT)],
                            deg_out.at[pl.ds(s * RPT, RPT)])

    return body(x0, x1, src_r, dst_r)


BN = 1000


def _tc_combine(aggsum, degbuf, x, wl_t, wr_t, b_row):
    def body(agg_ref, deg_ref, x_ref, wl_ref, wr_ref, b_ref, o_ref):
        inv = 1.0 / jnp.maximum(deg_ref[:, 0:1], 1.0)
        a0 = agg_ref[0] * inv
        a1 = agg_ref[1] * inv
        out = jnp.dot(a0, wl_ref[0:DH, :], preferred_element_type=jnp.float32)
        out = out + jnp.dot(a1, wl_ref[DH:D, :],
                            preferred_element_type=jnp.float32)
        out = out + jnp.dot(x_ref[...], wr_ref[...],
                            preferred_element_type=jnp.float32)
        out = out + b_ref[...]
        o_ref[...] = jnp.maximum(out, 0.0)

    return pl.pallas_call(
        body,
        grid=(N // BN,),
        in_specs=[
            pl.BlockSpec((NC, BN, DH), lambda i: (0, i, 0)),
            pl.BlockSpec((BN, 16), lambda i: (i, 0)),
            pl.BlockSpec((BN, D), lambda i: (i, 0)),
            pl.BlockSpec((D, D), lambda i: (0, 0)),
            pl.BlockSpec((D, D), lambda i: (0, 0)),
            pl.BlockSpec((1, D), lambda i: (0, 0)),
        ],
        out_specs=pl.BlockSpec((BN, D), lambda i: (i, 0)),
        out_shape=jax.ShapeDtypeStruct((N, D), jnp.float32),
    )(aggsum, degbuf, x, wl_t, wr_t, b_row)


@jax.jit
def kernel(x, edge_index, W_l, b_l, W_r):
    x0 = x[:, :DH]
    x1 = x[:, DH:]
    src_r = edge_index[0].reshape(NS, NCHUNK, CHUNK)
    dst_r = edge_index[1].reshape(NS, NCHUNK, CHUNK)
    aggsum, degbuf = _sc_aggregate(x0, x1, src_r, dst_r)
    return _tc_combine(aggsum, degbuf, x, W_l.T, W_r.T, b_l.reshape(1, D))
